# baseline (speedup 1.0000x reference)
import functools

import jax
import jax.numpy as jnp
from jax import lax
from jax.experimental import pallas as pl
from jax.experimental.pallas import tpu as pltpu
from jax.experimental.pallas import tpu_sc as plsc

_NUM_CORES = 2
_NUM_SUBCORES = 16
_NUM_WORKERS = _NUM_CORES * _NUM_SUBCORES
_CHUNK_ROWS = 16


def _make_sc_copy(seq_len, n_embd, dtype):
    rows_per_w = seq_len // _NUM_WORKERS
    n_chunks = rows_per_w // _CHUNK_ROWS
    mesh = plsc.VectorSubcoreMesh(
        core_axis_name="c", subcore_axis_name="s"
    )

    @functools.partial(
        pl.kernel,
        mesh=mesh,
        out_type=jax.ShapeDtypeStruct((seq_len, n_embd), dtype),
        scratch_types=[
            pltpu.VMEM((_CHUNK_ROWS, n_embd), dtype),
            pltpu.VMEM((_CHUNK_ROWS, n_embd), dtype),
            pltpu.SemaphoreType.DMA,
            pltpu.SemaphoreType.DMA,
            pltpu.SemaphoreType.DMA,
            pltpu.SemaphoreType.DMA,
        ],
    )
    def sc_copy(table_hbm, out_hbm, buf0, buf1, rs0, rs1, ws0, ws1):
        wid = lax.axis_index("s") * _NUM_CORES + lax.axis_index("c")
        base = wid * rows_per_w
        bufs = (buf0, buf1)
        rsems = (rs0, rs1)
        wsems = (ws0, ws1)

        def read(i):
            return pltpu.make_async_copy(
                table_hbm.at[pl.ds(base + i * _CHUNK_ROWS, _CHUNK_ROWS)],
                bufs[i % 2],
                rsems[i % 2],
            )

        def write(i):
            return pltpu.make_async_copy(
                bufs[i % 2],
                out_hbm.at[pl.ds(base + i * _CHUNK_ROWS, _CHUNK_ROWS)],
                wsems[i % 2],
            )

        read(0).start()
        for i in range(n_chunks):
            read(i).wait()
            write(i).start()
            if i + 1 < n_chunks:
                if i >= 1:
                    write(i - 1).wait()
                read(i + 1).start()
        write(n_chunks - 2).wait()
        write(n_chunks - 1).wait()

    return sc_copy


def kernel(x, emb_weight):
    seq_len = x.shape[1]
    n_embd = emb_weight.shape[1]
    return _make_sc_copy(seq_len, n_embd, emb_weight.dtype)(emb_weight)


# speedup vs baseline: 1.0253x; 1.0253x over previous
import functools

import jax
import jax.numpy as jnp
from jax import lax
from jax.experimental import pallas as pl
from jax.experimental.pallas import tpu as pltpu
from jax.experimental.pallas import tpu_sc as plsc

_NUM_CORES = 2
_NUM_SUBCORES = 16
_NUM_WORKERS = _NUM_CORES * _NUM_SUBCORES
_CHUNK_ROWS = 8
_NBUF = 4


def _make_sc_copy(seq_len, n_embd, dtype):
    rows_per_w = seq_len // _NUM_WORKERS
    n_chunks = rows_per_w // _CHUNK_ROWS
    mesh = plsc.VectorSubcoreMesh(
        core_axis_name="c", subcore_axis_name="s"
    )

    @functools.partial(
        pl.kernel,
        mesh=mesh,
        out_type=jax.ShapeDtypeStruct((seq_len, n_embd), dtype),
        scratch_types=(
            [pltpu.VMEM((_CHUNK_ROWS, n_embd), dtype)] * _NBUF
            + [pltpu.SemaphoreType.DMA] * (2 * _NBUF)
        ),
    )
    def sc_copy(table_hbm, out_hbm, *scratch):
        bufs = scratch[:_NBUF]
        rsems = scratch[_NBUF:2 * _NBUF]
        wsems = scratch[2 * _NBUF:]
        wid = lax.axis_index("s") * _NUM_CORES + lax.axis_index("c")
        base = wid * rows_per_w

        def read(i):
            return pltpu.make_async_copy(
                table_hbm.at[pl.ds(base + i * _CHUNK_ROWS, _CHUNK_ROWS)],
                bufs[i % _NBUF],
                rsems[i % _NBUF],
            )

        def write(i):
            return pltpu.make_async_copy(
                bufs[i % _NBUF],
                out_hbm.at[pl.ds(base + i * _CHUNK_ROWS, _CHUNK_ROWS)],
                wsems[i % _NBUF],
            )

        for j in range(min(_NBUF - 1, n_chunks)):
            read(j).start()
        waited = 0
        for i in range(n_chunks):
            read(i).wait()
            write(i).start()
            j = i + _NBUF - 1
            if j < n_chunks:
                if i >= 1:
                    write(i - 1).wait()
                    waited = i
                read(j).start()
        for i in range(waited, n_chunks):
            write(i).wait()

    return sc_copy


def kernel(x, emb_weight):
    seq_len = x.shape[1]
    n_embd = emb_weight.shape[1]
    return _make_sc_copy(seq_len, n_embd, emb_weight.dtype)(emb_weight)


# speedup vs baseline: 1.0348x; 1.0093x over previous
import functools

import jax
import jax.numpy as jnp
from jax import lax
from jax.experimental import pallas as pl
from jax.experimental.pallas import tpu as pltpu
from jax.experimental.pallas import tpu_sc as plsc

_NUM_CORES = 2
_NUM_SUBCORES = 16
_NUM_WORKERS = _NUM_CORES * _NUM_SUBCORES
_CHUNK_ROWS = 16
_NBUF = 3


def _make_sc_copy(seq_len, n_embd, dtype):
    rows_per_w = seq_len // _NUM_WORKERS
    n_chunks = rows_per_w // _CHUNK_ROWS
    mesh = plsc.VectorSubcoreMesh(
        core_axis_name="c", subcore_axis_name="s"
    )

    @functools.partial(
        pl.kernel,
        mesh=mesh,
        out_type=jax.ShapeDtypeStruct((seq_len, n_embd), dtype),
        scratch_types=(
            [pltpu.VMEM((_CHUNK_ROWS, n_embd), dtype)] * _NBUF
            + [pltpu.SemaphoreType.DMA] * (2 * _NBUF)
        ),
    )
    def sc_copy(table_hbm, out_hbm, *scratch):
        bufs = scratch[:_NBUF]
        rsems = scratch[_NBUF:2 * _NBUF]
        wsems = scratch[2 * _NBUF:]
        wid = lax.axis_index("s") * _NUM_CORES + lax.axis_index("c")
        base = wid * rows_per_w

        def read(i):
            return pltpu.make_async_copy(
                table_hbm.at[pl.ds(base + i * _CHUNK_ROWS, _CHUNK_ROWS)],
                bufs[i % _NBUF],
                rsems[i % _NBUF],
            )

        def write(i):
            return pltpu.make_async_copy(
                bufs[i % _NBUF],
                out_hbm.at[pl.ds(base + i * _CHUNK_ROWS, _CHUNK_ROWS)],
                wsems[i % _NBUF],
            )

        for j in range(min(_NBUF - 1, n_chunks)):
            read(j).start()
        waited = 0
        for i in range(n_chunks):
            read(i).wait()
            write(i).start()
            j = i + _NBUF - 1
            if j < n_chunks:
                if i >= 1:
                    write(i - 1).wait()
                    waited = i
                read(j).start()
        for i in range(waited, n_chunks):
            write(i).wait()

    return sc_copy


def kernel(x, emb_weight):
    seq_len = x.shape[1]
    n_embd = emb_weight.shape[1]
    return _make_sc_copy(seq_len, n_embd, emb_weight.dtype)(emb_weight)


# speedup vs baseline: 1.0603x; 1.0246x over previous
import functools

import jax
import jax.numpy as jnp
from jax import lax
from jax.experimental import pallas as pl
from jax.experimental.pallas import tpu as pltpu
from jax.experimental.pallas import tpu_sc as plsc

_NUM_CORES = 2
_NUM_SUBCORES = 16
_NUM_WORKERS = _NUM_CORES * _NUM_SUBCORES
_CHUNK_ROWS = 16
_NBUF = 3


def _make_sc_copy(seq_len, n_embd, dtype):
    rows_per_w = seq_len // _NUM_WORKERS
    n_chunks = rows_per_w // _CHUNK_ROWS
    mesh = plsc.VectorSubcoreMesh(
        core_axis_name="c", subcore_axis_name="s"
    )

    @functools.partial(
        pl.kernel,
        mesh=mesh,
        out_type=jax.ShapeDtypeStruct((seq_len, n_embd), dtype),
        scratch_types=(
            [pltpu.VMEM_SHARED((_NBUF, _NUM_SUBCORES, _CHUNK_ROWS, n_embd), dtype)]
            + [pltpu.SemaphoreType.DMA] * (2 * _NBUF)
        ),
    )
    def sc_copy(table_hbm, out_hbm, spbuf, *sems):
        rsems = sems[:_NBUF]
        wsems = sems[_NBUF:]
        s = lax.axis_index("s")
        wid = s * _NUM_CORES + lax.axis_index("c")
        base = wid * rows_per_w

        def read(i):
            return pltpu.make_async_copy(
                table_hbm.at[pl.ds(base + i * _CHUNK_ROWS, _CHUNK_ROWS)],
                spbuf.at[i % _NBUF, s],
                rsems[i % _NBUF],
            )

        def write(i):
            return pltpu.make_async_copy(
                spbuf.at[i % _NBUF, s],
                out_hbm.at[pl.ds(base + i * _CHUNK_ROWS, _CHUNK_ROWS)],
                wsems[i % _NBUF],
            )

        for j in range(min(_NBUF - 1, n_chunks)):
            read(j).start()
        waited = 0
        for i in range(n_chunks):
            read(i).wait()
            write(i).start()
            j = i + _NBUF - 1
            if j < n_chunks:
                if i >= 1:
                    write(i - 1).wait()
                    waited = i
                read(j).start()
        for i in range(waited, n_chunks):
            write(i).wait()

    return sc_copy


def kernel(x, emb_weight):
    seq_len = x.shape[1]
    n_embd = emb_weight.shape[1]
    return _make_sc_copy(seq_len, n_embd, emb_weight.dtype)(emb_weight)
